# baseline (speedup 1.0000x reference)
import jax
import jax.numpy as jnp
from jax import lax
from jax.experimental import pallas as pl
from jax.experimental.pallas import tpu as pltpu
from jax.experimental.pallas import tpu_sc as plsc

_B, _C, _H, _W, _K = 2, 96, 224, 224, 2
_CP = 128
_HW = _H * _W
_N = _B * _HW
_NC, _NS, _L = 2, 16, 16
_NW = _NC * _NS
_ROWS_PER = _N // _NW
_CHUNK = 64
_NCHUNK = _ROWS_PER // _CHUNK


def _body(inp_hbm, flow_hbm, att_hbm, out_hbm,
          fx, fy, av, ia, ib, ic, id_, wa, wb, wc, wd,
          ra, rb, rc, rd, ov, sem0, sem1):
    wid = lax.axis_index("s") * _NC + lax.axis_index("c")
    b = wid // 16
    wloc = wid - b * 16
    imgbase = b * _HW
    fbase0 = b * (2 * _K * _HW)
    abase0 = b * (_K * _HW)

    idx_s = [[ia.at[s], ib.at[s], ic.at[s], id_.at[s]] for s in range(2)]
    w_s = [[wa.at[s], wb.at[s], wc.at[s], wd.at[s]] for s in range(2)]
    rows_s = [[ra.at[s], rb.at[s], rc.at[s], rd.at[s]] for s in range(2)]
    sems = [sem0, sem1]

    def stage_and_fire(ci, k, s):
        off = wloc * _ROWS_PER + ci * _CHUNK
        pltpu.sync_copy(flow_hbm.at[pl.ds(fbase0 + 2 * k * _HW + off, _CHUNK)], fx)
        pltpu.sync_copy(flow_hbm.at[pl.ds(fbase0 + (2 * k + 1) * _HW + off, _CHUNK)], fy)
        pltpu.sync_copy(att_hbm.at[pl.ds(abase0 + k * _HW + off, _CHUNK)], av)
        yrow0 = off // _W
        xb = off - yrow0 * _W
        for j in range(_CHUNK // _L):
            sl = pl.ds(j * _L, _L)
            xc_raw = xb + j * _L + lax.iota(jnp.int32, _L)
            wrap = xc_raw >= _W
            xc = jnp.where(wrap, xc_raw - _W, xc_raw)
            yc = yrow0 + jnp.where(wrap, 1, 0)
            x = jnp.clip(xc.astype(jnp.float32) + fx[sl], 0.0, _W - 1.0)
            y = jnp.clip(yc.astype(jnp.float32) + fy[sl], 0.0, _H - 1.0)
            x0 = x.astype(jnp.int32)
            y0 = y.astype(jnp.int32)
            dx = x - x0.astype(jnp.float32)
            dy = y - y0.astype(jnp.float32)
            x1 = jnp.minimum(x0 + 1, _W - 1)
            y1 = jnp.minimum(y0 + 1, _H - 1)
            ry0 = imgbase + y0 * _W
            ry1 = imgbase + y1 * _W
            idx_s[s][0][sl] = ry0 + x0
            idx_s[s][1][sl] = ry1 + x0
            idx_s[s][2][sl] = ry0 + x1
            idx_s[s][3][sl] = ry1 + x1
            a_v = av[sl]
            omdx = 1.0 - dx
            omdy = 1.0 - dy
            w_s[s][0][sl] = omdx * omdy * a_v
            w_s[s][1][sl] = omdx * dy * a_v
            w_s[s][2][sl] = dx * omdy * a_v
            w_s[s][3][sl] = dx * dy * a_v
        for t in range(4):
            pltpu.async_copy(inp_hbm.at[idx_s[s][t]], rows_s[s][t], sems[s])

    def drain(s):
        for t in range(4):
            pltpu.make_async_copy(inp_hbm.at[idx_s[s][t]], rows_s[s][t],
                                  sems[s]).wait()

    def fma(s, accumulate):
        def fma_body(p, _):
            pv = jnp.full((_L,), p, dtype=jnp.int32)
            ws = [plsc.load_gather(w_s[s][t], [pv]) for t in range(4)]
            for cc in range(_C // _L):
                cs = pl.ds(cc * _L, _L)
                contrib = (ws[0] * rows_s[s][0][p, cs] +
                           ws[1] * rows_s[s][1][p, cs] +
                           ws[2] * rows_s[s][2][p, cs] +
                           ws[3] * rows_s[s][3][p, cs])
                if accumulate:
                    ov[p, cs] = ov[p, cs] + contrib
                else:
                    ov[p, cs] = contrib
            return _

        lax.fori_loop(0, _CHUNK, fma_body, None)

    stage_and_fire(0, 0, 0)

    def chunk_body(ci, carry):
        stage_and_fire(ci, 1, 1)
        drain(0)
        fma(0, accumulate=False)
        cin = jnp.minimum(ci + 1, _NCHUNK - 1)
        stage_and_fire(cin, 0, 0)
        drain(1)
        fma(1, accumulate=True)
        off = wloc * _ROWS_PER + ci * _CHUNK
        pltpu.sync_copy(ov, out_hbm.at[pl.ds(imgbase + off, _CHUNK), :])
        return carry

    lax.fori_loop(0, _NCHUNK, chunk_body, None)
    drain(0)


def _tp_body(x_ref, o_ref):
    x = x_ref[0]
    xt = jnp.transpose(x, (1, 2, 0))
    o_ref[:, :_C] = xt.reshape(8 * _W, _C)
    o_ref[:, _C:] = jnp.zeros((8 * _W, _CP - _C), jnp.float32)


def _transpose_pad(x):
    return pl.pallas_call(
        _tp_body,
        out_shape=jax.ShapeDtypeStruct((_N, _CP), jnp.float32),
        grid=(_B, _H // 8),
        in_specs=[pl.BlockSpec((1, _C, 8, _W), lambda b, h: (b, 0, h, 0))],
        out_specs=pl.BlockSpec((8 * _W, _CP), lambda b, h: (b * (_H // 8) + h, 0)),
    )(x)


def _warp_sc(inp_t, flow_r, att_r):
    mesh = plsc.VectorSubcoreMesh(core_axis_name="c", subcore_axis_name="s")
    return pl.kernel(
        _body,
        out_type=jax.ShapeDtypeStruct((_N, _C), jnp.float32),
        mesh=mesh,
        compiler_params=pltpu.CompilerParams(needs_layout_passes=False),
        scratch_types=[
            pltpu.VMEM((_CHUNK,), jnp.float32),
            pltpu.VMEM((_CHUNK,), jnp.float32),
            pltpu.VMEM((_CHUNK,), jnp.float32),
            pltpu.VMEM((2, _CHUNK), jnp.int32),
            pltpu.VMEM((2, _CHUNK), jnp.int32),
            pltpu.VMEM((2, _CHUNK), jnp.int32),
            pltpu.VMEM((2, _CHUNK), jnp.int32),
            pltpu.VMEM((2, _CHUNK), jnp.float32),
            pltpu.VMEM((2, _CHUNK), jnp.float32),
            pltpu.VMEM((2, _CHUNK), jnp.float32),
            pltpu.VMEM((2, _CHUNK), jnp.float32),
            pltpu.VMEM((2, _CHUNK, _CP), jnp.float32),
            pltpu.VMEM((2, _CHUNK, _CP), jnp.float32),
            pltpu.VMEM((2, _CHUNK, _CP), jnp.float32),
            pltpu.VMEM((2, _CHUNK, _CP), jnp.float32),
            pltpu.VMEM((_CHUNK, _C), jnp.float32),
            pltpu.SemaphoreType.DMA,
            pltpu.SemaphoreType.DMA,
        ],
    )(inp_t, flow_r, att_r)


def kernel(input, flow, attention):
    inp_t = _transpose_pad(input)
    flow_r = flow.reshape(_B * 2 * _K * _HW)
    att_r = attention.reshape(_B * _K * _HW)
    out_t = _warp_sc(inp_t, flow_r, att_r)
    return jnp.transpose(out_t.reshape(_B, _H, _W, _C), (0, 3, 1, 2))


# speedup vs baseline: 1.4140x; 1.4140x over previous
import jax
import jax.numpy as jnp
from jax import lax
from jax.experimental import pallas as pl
from jax.experimental.pallas import tpu as pltpu
from jax.experimental.pallas import tpu_sc as plsc

_B, _C, _H, _W, _K = 2, 96, 224, 224, 2
_CP = 128
_HW = _H * _W
_N = _B * _HW
_NC, _NS, _L = 2, 16, 16
_NW = _NC * _NS
_ROWS_PER = _N // _NW
_CHUNK = 64
_NCHUNK = _ROWS_PER // _CHUNK


def _body(inp_hbm, flow_hbm, att_hbm, out_hbm,
          fxw, fyw, avw, ia, ib, ic, id_, wa, wb, wc, wd,
          ra, rb, rc, rd, ov, sem0, sem1):
    wid = lax.axis_index("s") * _NC + lax.axis_index("c")
    b = wid // 16
    wloc = wid - b * 16
    imgbase = b * _HW
    fbase0 = b * (2 * _K * _HW)
    abase0 = b * (_K * _HW)
    loff = wloc * _ROWS_PER

    for k in range(_K):
        pltpu.sync_copy(flow_hbm.at[pl.ds(fbase0 + 2 * k * _HW + loff, _ROWS_PER)],
                        fxw.at[pl.ds(k * _ROWS_PER, _ROWS_PER)])
        pltpu.sync_copy(flow_hbm.at[pl.ds(fbase0 + (2 * k + 1) * _HW + loff, _ROWS_PER)],
                        fyw.at[pl.ds(k * _ROWS_PER, _ROWS_PER)])
        pltpu.sync_copy(att_hbm.at[pl.ds(abase0 + k * _HW + loff, _ROWS_PER)],
                        avw.at[pl.ds(k * _ROWS_PER, _ROWS_PER)])

    idx_s = [[ia.at[s], ib.at[s], ic.at[s], id_.at[s]] for s in range(2)]
    w_s = [[wa.at[s], wb.at[s], wc.at[s], wd.at[s]] for s in range(2)]
    rows_s = [[ra.at[s], rb.at[s], rc.at[s], rd.at[s]] for s in range(2)]
    sems = [sem0, sem1]

    def stage_and_fire(ci, k, s):
        lo = ci * _CHUNK
        off = loff + lo
        yrow0 = off // _W
        xb = off - yrow0 * _W
        for j in range(_CHUNK // _L):
            sl = pl.ds(j * _L, _L)
            lsl = pl.ds(k * _ROWS_PER + lo + j * _L, _L)
            xc_raw = xb + j * _L + lax.iota(jnp.int32, _L)
            wrap = xc_raw >= _W
            xc = jnp.where(wrap, xc_raw - _W, xc_raw)
            yc = yrow0 + jnp.where(wrap, 1, 0)
            x = jnp.clip(xc.astype(jnp.float32) + fxw[lsl], 0.0, _W - 1.0)
            y = jnp.clip(yc.astype(jnp.float32) + fyw[lsl], 0.0, _H - 1.0)
            x0 = x.astype(jnp.int32)
            y0 = y.astype(jnp.int32)
            dx = x - x0.astype(jnp.float32)
            dy = y - y0.astype(jnp.float32)
            x1 = jnp.minimum(x0 + 1, _W - 1)
            y1 = jnp.minimum(y0 + 1, _H - 1)
            ry0 = imgbase + y0 * _W
            ry1 = imgbase + y1 * _W
            idx_s[s][0][sl] = ry0 + x0
            idx_s[s][1][sl] = ry1 + x0
            idx_s[s][2][sl] = ry0 + x1
            idx_s[s][3][sl] = ry1 + x1
            a_v = avw[lsl]
            omdx = 1.0 - dx
            omdy = 1.0 - dy
            w_s[s][0][sl] = omdx * omdy * a_v
            w_s[s][1][sl] = omdx * dy * a_v
            w_s[s][2][sl] = dx * omdy * a_v
            w_s[s][3][sl] = dx * dy * a_v
        for t in range(4):
            pltpu.async_copy(inp_hbm.at[idx_s[s][t]], rows_s[s][t], sems[s])

    def drain(s):
        for t in range(4):
            pltpu.make_async_copy(inp_hbm.at[idx_s[s][t]], rows_s[s][t],
                                  sems[s]).wait()

    def fma(s, accumulate):
        def fma_body(p, _):
            pv = jnp.full((_L,), p, dtype=jnp.int32)
            ws = [plsc.load_gather(w_s[s][t], [pv]) for t in range(4)]
            for cc in range(_C // _L):
                cs = pl.ds(cc * _L, _L)
                contrib = (ws[0] * rows_s[s][0][p, cs] +
                           ws[1] * rows_s[s][1][p, cs] +
                           ws[2] * rows_s[s][2][p, cs] +
                           ws[3] * rows_s[s][3][p, cs])
                if accumulate:
                    ov[p, cs] = ov[p, cs] + contrib
                else:
                    ov[p, cs] = contrib
            return _

        lax.fori_loop(0, _CHUNK, fma_body, None)

    stage_and_fire(0, 0, 0)

    def chunk_body(ci, carry):
        stage_and_fire(ci, 1, 1)
        drain(0)
        fma(0, accumulate=False)
        cin = jnp.minimum(ci + 1, _NCHUNK - 1)
        stage_and_fire(cin, 0, 0)
        drain(1)
        fma(1, accumulate=True)
        off = wloc * _ROWS_PER + ci * _CHUNK
        pltpu.sync_copy(ov, out_hbm.at[pl.ds(imgbase + off, _CHUNK), :])
        return carry

    lax.fori_loop(0, _NCHUNK, chunk_body, None)
    drain(0)


def _warp_sc(inp_t, flow_r, att_r):
    mesh = plsc.VectorSubcoreMesh(core_axis_name="c", subcore_axis_name="s")
    return pl.kernel(
        _body,
        out_type=jax.ShapeDtypeStruct((_N, _C), jnp.float32),
        mesh=mesh,
        compiler_params=pltpu.CompilerParams(needs_layout_passes=False),
        scratch_types=[
            pltpu.VMEM((_K * _ROWS_PER,), jnp.float32),
            pltpu.VMEM((_K * _ROWS_PER,), jnp.float32),
            pltpu.VMEM((_K * _ROWS_PER,), jnp.float32),
            pltpu.VMEM((2, _CHUNK), jnp.int32),
            pltpu.VMEM((2, _CHUNK), jnp.int32),
            pltpu.VMEM((2, _CHUNK), jnp.int32),
            pltpu.VMEM((2, _CHUNK), jnp.int32),
            pltpu.VMEM((2, _CHUNK), jnp.float32),
            pltpu.VMEM((2, _CHUNK), jnp.float32),
            pltpu.VMEM((2, _CHUNK), jnp.float32),
            pltpu.VMEM((2, _CHUNK), jnp.float32),
            pltpu.VMEM((2, _CHUNK, _CP), jnp.float32),
            pltpu.VMEM((2, _CHUNK, _CP), jnp.float32),
            pltpu.VMEM((2, _CHUNK, _CP), jnp.float32),
            pltpu.VMEM((2, _CHUNK, _CP), jnp.float32),
            pltpu.VMEM((_CHUNK, _C), jnp.float32),
            pltpu.SemaphoreType.DMA,
            pltpu.SemaphoreType.DMA,
        ],
    )(inp_t, flow_r, att_r)


def kernel(input, flow, attention):
    inp_t = jnp.transpose(input, (0, 2, 3, 1)).reshape(_N, _C)
    inp_t = jnp.pad(inp_t, ((0, 0), (0, _CP - _C)))
    flow_r = flow.reshape(_B * 2 * _K * _HW)
    att_r = attention.reshape(_B * _K * _HW)
    out_t = _warp_sc(inp_t, flow_r, att_r)
    return jnp.transpose(out_t.reshape(_B, _H, _W, _C), (0, 3, 1, 2))


# speedup vs baseline: 1.6813x; 1.1890x over previous
import jax
import jax.numpy as jnp
from jax import lax
from jax.experimental import pallas as pl
from jax.experimental.pallas import tpu as pltpu
from jax.experimental.pallas import tpu_sc as plsc

_B, _C, _H, _W, _K = 2, 96, 224, 224, 2
_CP = 128
_HW = _H * _W
_N = _B * _HW
_NC, _NS, _L = 2, 16, 16
_NW = _NC * _NS
_ROWS_PER = _N // _NW
_CHUNK = 64
_NCHUNK = _ROWS_PER // _CHUNK


def _body(inp_hbm, flow_hbm, att_hbm, out_hbm,
          fxw, fyw, avw, ia, ib, ic, id_, wa, wb, wc, wd,
          ra, rb, rc, rd, ov, sem0, sem1, sem_out):
    wid = lax.axis_index("s") * _NC + lax.axis_index("c")
    b = wid // 16
    wloc = wid - b * 16
    imgbase = b * _HW
    fbase0 = b * (2 * _K * _HW)
    abase0 = b * (_K * _HW)
    loff = wloc * _ROWS_PER

    for k in range(_K):
        pltpu.sync_copy(flow_hbm.at[pl.ds(fbase0 + 2 * k * _HW + loff, _ROWS_PER)],
                        fxw.at[pl.ds(k * _ROWS_PER, _ROWS_PER)])
        pltpu.sync_copy(flow_hbm.at[pl.ds(fbase0 + (2 * k + 1) * _HW + loff, _ROWS_PER)],
                        fyw.at[pl.ds(k * _ROWS_PER, _ROWS_PER)])
        pltpu.sync_copy(att_hbm.at[pl.ds(abase0 + k * _HW + loff, _ROWS_PER)],
                        avw.at[pl.ds(k * _ROWS_PER, _ROWS_PER)])

    idx_s = [[ia.at[s], ib.at[s], ic.at[s], id_.at[s]] for s in range(2)]
    w_s = [[wa.at[s], wb.at[s], wc.at[s], wd.at[s]] for s in range(2)]
    rows_s = [[ra.at[s], rb.at[s], rc.at[s], rd.at[s]] for s in range(2)]
    sems = [sem0, sem1]

    def stage_and_fire(ci, k, s):
        lo = ci * _CHUNK
        off = loff + lo
        yrow0 = off // _W
        xb = off - yrow0 * _W
        for j in range(_CHUNK // _L):
            sl = pl.ds(j * _L, _L)
            lsl = pl.ds(k * _ROWS_PER + lo + j * _L, _L)
            xc_raw = xb + j * _L + lax.iota(jnp.int32, _L)
            wrap = xc_raw >= _W
            xc = jnp.where(wrap, xc_raw - _W, xc_raw)
            yc = yrow0 + jnp.where(wrap, 1, 0)
            x = jnp.clip(xc.astype(jnp.float32) + fxw[lsl], 0.0, _W - 1.0)
            y = jnp.clip(yc.astype(jnp.float32) + fyw[lsl], 0.0, _H - 1.0)
            x0 = x.astype(jnp.int32)
            y0 = y.astype(jnp.int32)
            dx = x - x0.astype(jnp.float32)
            dy = y - y0.astype(jnp.float32)
            x1 = jnp.minimum(x0 + 1, _W - 1)
            y1 = jnp.minimum(y0 + 1, _H - 1)
            ry0 = imgbase + y0 * _W
            ry1 = imgbase + y1 * _W
            idx_s[s][0][sl] = ry0 + x0
            idx_s[s][1][sl] = ry1 + x0
            idx_s[s][2][sl] = ry0 + x1
            idx_s[s][3][sl] = ry1 + x1
            a_v = avw[lsl]
            omdx = 1.0 - dx
            omdy = 1.0 - dy
            w_s[s][0][sl] = omdx * omdy * a_v
            w_s[s][1][sl] = omdx * dy * a_v
            w_s[s][2][sl] = dx * omdy * a_v
            w_s[s][3][sl] = dx * dy * a_v
        for t in range(4):
            pltpu.async_copy(inp_hbm.at[idx_s[s][t]], rows_s[s][t], sems[s])

    def drain(s):
        for t in range(4):
            pltpu.make_async_copy(inp_hbm.at[idx_s[s][t]], rows_s[s][t],
                                  sems[s]).wait()

    def fma(s, accumulate):
        def fma_body(q, _):
            for u in range(2):
                p = q * 2 + u
                pv = jnp.full((_L,), p, dtype=jnp.int32)
                ws = [plsc.load_gather(w_s[s][t], [pv]) for t in range(4)]
                for cc in range(_C // _L):
                    cs = pl.ds(cc * _L, _L)
                    contrib = (ws[0] * rows_s[s][0][p, cs] +
                               ws[1] * rows_s[s][1][p, cs] +
                               ws[2] * rows_s[s][2][p, cs] +
                               ws[3] * rows_s[s][3][p, cs])
                    if accumulate:
                        ov[p, cs] = ov[p, cs] + contrib
                    else:
                        ov[p, cs] = contrib
            return _

        lax.fori_loop(0, _CHUNK // 2, fma_body, None)

    stage_and_fire(0, 0, 0)

    def chunk_body(ci, carry):
        stage_and_fire(ci, 1, 1)
        drain(0)
        off = wloc * _ROWS_PER + ci * _CHUNK
        dst = out_hbm.at[pl.ds(imgbase + off, _CHUNK), :]

        @pl.when(ci > 0)
        def _wait_prev_out():
            pltpu.make_async_copy(ov, dst, sem_out).wait()

        fma(0, accumulate=False)
        cin = jnp.minimum(ci + 1, _NCHUNK - 1)
        stage_and_fire(cin, 0, 0)
        drain(1)
        fma(1, accumulate=True)
        pltpu.async_copy(ov, dst, sem_out)
        return carry

    lax.fori_loop(0, _NCHUNK, chunk_body, None)
    drain(0)
    lastdst = out_hbm.at[pl.ds(imgbase + loff + (_NCHUNK - 1) * _CHUNK, _CHUNK), :]
    pltpu.make_async_copy(ov, lastdst, sem_out).wait()


def _warp_sc(inp_t, flow_r, att_r):
    mesh = plsc.VectorSubcoreMesh(core_axis_name="c", subcore_axis_name="s")
    return pl.kernel(
        _body,
        out_type=jax.ShapeDtypeStruct((_N, _C), jnp.float32),
        mesh=mesh,
        compiler_params=pltpu.CompilerParams(needs_layout_passes=False),
        scratch_types=[
            pltpu.VMEM((_K * _ROWS_PER,), jnp.float32),
            pltpu.VMEM((_K * _ROWS_PER,), jnp.float32),
            pltpu.VMEM((_K * _ROWS_PER,), jnp.float32),
            pltpu.VMEM((2, _CHUNK), jnp.int32),
            pltpu.VMEM((2, _CHUNK), jnp.int32),
            pltpu.VMEM((2, _CHUNK), jnp.int32),
            pltpu.VMEM((2, _CHUNK), jnp.int32),
            pltpu.VMEM((2, _CHUNK), jnp.float32),
            pltpu.VMEM((2, _CHUNK), jnp.float32),
            pltpu.VMEM((2, _CHUNK), jnp.float32),
            pltpu.VMEM((2, _CHUNK), jnp.float32),
            pltpu.VMEM((2, _CHUNK, _CP), jnp.float32),
            pltpu.VMEM((2, _CHUNK, _CP), jnp.float32),
            pltpu.VMEM((2, _CHUNK, _CP), jnp.float32),
            pltpu.VMEM((2, _CHUNK, _CP), jnp.float32),
            pltpu.VMEM((_CHUNK, _C), jnp.float32),
            pltpu.SemaphoreType.DMA,
            pltpu.SemaphoreType.DMA,
            pltpu.SemaphoreType.DMA,
        ],
    )(inp_t, flow_r, att_r)


def kernel(input, flow, attention):
    inp_t = jnp.transpose(input, (0, 2, 3, 1)).reshape(_N, _C)
    inp_t = jnp.pad(inp_t, ((0, 0), (0, _CP - _C)))
    flow_r = flow.reshape(_B * 2 * _K * _HW)
    att_r = attention.reshape(_B * _K * _HW)
    out_t = _warp_sc(inp_t, flow_r, att_r)
    return jnp.transpose(out_t.reshape(_B, _H, _W, _C), (0, 3, 1, 2))
